# baseline (speedup 1.0000x reference)
import functools

import jax
import jax.numpy as jnp
from jax import lax
from jax.experimental import pallas as pl
from jax.experimental.pallas import tpu as pltpu
from jax.experimental.pallas import tpu_sc as plsc

B, T, K, H = 1024, 10, 16, 256
NW = 32
GROUPS = B * T
GPW = GROUPS // NW
SPW = B // NW
GB = 8
RB = GB * K
NBATCH = GPW // GB
LANES = 16
C = H // LANES

_MESH = plsc.VectorSubcoreMesh(core_axis_name="c", subcore_axis_name="s")


def _sc_gather_sum(hist3, s, r, ent, rel):
    @functools.partial(
        pl.kernel,
        mesh=_MESH,
        out_type=[
            jax.ShapeDtypeStruct((GROUPS, H), jnp.float32),
            jax.ShapeDtypeStruct((B, H), jnp.float32),
            jax.ShapeDtypeStruct((B, H), jnp.float32),
        ],
        scratch_types=[
            pltpu.VMEM((T, K), jnp.int32),
            pltpu.VMEM((GPW * K,), jnp.int32),
            pltpu.VMEM((RB, H), jnp.float32),
            pltpu.VMEM((RB, H), jnp.float32),
            pltpu.VMEM((GB, H), jnp.float32),
            pltpu.VMEM((GB, H), jnp.float32),
            pltpu.VMEM((SPW,), jnp.int32),
            pltpu.VMEM((SPW, H), jnp.float32),
            pltpu.SemaphoreType.DMA,
            pltpu.SemaphoreType.DMA,
            pltpu.SemaphoreType.DMA,
            pltpu.SemaphoreType.DMA,
        ],
    )
    def k(hist_hbm, s_hbm, r_hbm, ent_hbm, rel_hbm,
          sums_hbm, se_hbm, re_hbm,
          idx2_v, idx_v, rows0, rows1, sums0, sums1, sidx_v, srows_v,
          sem0, sem1, semw0, semw1):
        wid = lax.axis_index("s") * 2 + lax.axis_index("c")
        base_s = wid * SPW
        base_g = wid * GPW

        def flat_s(si, carry):
            pltpu.sync_copy(hist_hbm.at[base_s + si], idx2_v)

            def flat_t(ti, carry2):
                idx_v[pl.ds((si * T + ti) * K, K)] = idx2_v[ti]
                return carry2
            return lax.fori_loop(0, T, flat_t, carry)

        lax.fori_loop(0, SPW, flat_s, 0)

        def start_gather(bi, rows_ref, sem):
            pltpu.make_async_copy(
                ent_hbm.at[idx_v.at[pl.ds(bi * RB, RB)]], rows_ref, sem
            ).start()

        def sums_copy(bi, sums_ref, semw):
            return pltpu.make_async_copy(
                sums_ref, sums_hbm.at[pl.ds(base_g + bi * GB, GB)], semw)

        def finish_batch(j, bi, rows_ref, sem, sums_ref, semw):
            pltpu.make_async_copy(
                ent_hbm.at[idx_v.at[pl.ds(bi * RB, RB)]], rows_ref, sem
            ).wait()

            @pl.when(j > 0)
            def _():
                sums_copy(bi - 2, sums_ref, semw).wait()

            def group_body(g, carry):
                for c in range(C):
                    vs = [rows_ref[g * K + rr, pl.ds(c * LANES, LANES)]
                          for rr in range(K)]
                    while len(vs) > 1:
                        vs = [vs[i] + vs[i + 1] for i in range(0, len(vs), 2)]
                    sums_ref[g, pl.ds(c * LANES, LANES)] = vs[0]
                return carry

            lax.fori_loop(0, GB, group_body, 0)
            sums_copy(bi, sums_ref, semw).start()

        start_gather(0, rows0, sem0)

        def outer(j, carry):
            start_gather(2 * j + 1, rows1, sem1)
            finish_batch(j, 2 * j, rows0, sem0, sums0, semw0)

            @pl.when(j < NBATCH // 2 - 1)
            def _():
                start_gather(2 * j + 2, rows0, sem0)

            finish_batch(j, 2 * j + 1, rows1, sem1, sums1, semw1)
            return carry

        lax.fori_loop(0, NBATCH // 2, outer, 0)

        sums_copy(NBATCH - 2, sums0, semw0).wait()
        sums_copy(NBATCH - 1, sums1, semw1).wait()

        pltpu.sync_copy(s_hbm.at[pl.ds(base_s, SPW)], sidx_v)
        pltpu.async_copy(ent_hbm.at[sidx_v], srows_v, sem0).wait()
        pltpu.sync_copy(srows_v, se_hbm.at[pl.ds(base_s, SPW)])
        pltpu.sync_copy(r_hbm.at[pl.ds(base_s, SPW)], sidx_v)
        pltpu.async_copy(rel_hbm.at[sidx_v], srows_v, sem0).wait()
        pltpu.sync_copy(srows_v, re_hbm.at[pl.ds(base_s, SPW)])

    return k(hist3, s, r, ent, rel)


def _tc_finish(sums2, s_e, r_e, W, b2):
    BB = 64

    def body(sums_ref, se_ref, re_ref, w_ref, b_ref, out_ref):
        x = sums_ref[...] * (1.0 / K)
        y = jnp.dot(x, w_ref[...], preferred_element_type=jnp.float32)
        y = jnp.maximum(y + b_ref[...], 0.0)
        out_ref[:, :, 0:H] = y.reshape(BB, T, H)
        out_ref[:, :, H:2 * H] = jnp.broadcast_to(
            se_ref[...][:, None, :], (BB, T, H))
        out_ref[:, :, 2 * H:3 * H] = jnp.broadcast_to(
            re_ref[...][:, None, :], (BB, T, H))

    return pl.pallas_call(
        body,
        grid=(B // BB,),
        in_specs=[
            pl.BlockSpec((BB * T, H), lambda i: (i, 0)),
            pl.BlockSpec((BB, H), lambda i: (i, 0)),
            pl.BlockSpec((BB, H), lambda i: (i, 0)),
            pl.BlockSpec((H, H), lambda i: (0, 0)),
            pl.BlockSpec((1, H), lambda i: (0, 0)),
        ],
        out_specs=pl.BlockSpec((BB, T, 3 * H), lambda i: (i, 0, 0)),
        out_shape=jax.ShapeDtypeStruct((B, T, 3 * H), jnp.float32),
    )(sums2, s_e, r_e, W, b2)


def kernel(s_hist, s, r, ent_embeds, rel_embeds, W, b):
    hist3 = s_hist.astype(jnp.int32)
    sums, s_e, r_e = _sc_gather_sum(
        hist3, s.astype(jnp.int32), r.astype(jnp.int32),
        ent_embeds, rel_embeds)
    return _tc_finish(sums, s_e, r_e, W, b.reshape(1, H))


# speedup vs baseline: 1.0838x; 1.0838x over previous
import functools

import jax
import jax.numpy as jnp
from jax import lax
from jax.experimental import pallas as pl
from jax.experimental.pallas import tpu as pltpu
from jax.experimental.pallas import tpu_sc as plsc

B, T, K, H = 1024, 10, 16, 256
NW = 32
GROUPS = B * T
GPW = GROUPS // NW
SPW = B // NW
GB = 8
RB = GB * K
NBATCH = GPW // GB
LANES = 16
C = H // LANES

_MESH = plsc.VectorSubcoreMesh(core_axis_name="c", subcore_axis_name="s")


def _sc_gather_sum(hist, s, r, ent, rel):
    @functools.partial(
        pl.kernel,
        mesh=_MESH,
        out_type=[
            jax.ShapeDtypeStruct((GROUPS, H), jnp.float32),
            jax.ShapeDtypeStruct((B, H), jnp.float32),
            jax.ShapeDtypeStruct((B, H), jnp.float32),
        ],
        scratch_types=[
            pltpu.VMEM((GPW * K,), jnp.int32),
            pltpu.VMEM((RB, H), jnp.float32),
            pltpu.VMEM((RB, H), jnp.float32),
            pltpu.VMEM((GB, H), jnp.float32),
            pltpu.VMEM((GB, H), jnp.float32),
            pltpu.VMEM((SPW,), jnp.int32),
            pltpu.VMEM((SPW,), jnp.int32),
            pltpu.VMEM((SPW, H), jnp.float32),
            pltpu.VMEM((SPW, H), jnp.float32),
            pltpu.SemaphoreType.DMA,
            pltpu.SemaphoreType.DMA,
            pltpu.SemaphoreType.DMA,
            pltpu.SemaphoreType.DMA,
            pltpu.SemaphoreType.DMA,
            pltpu.SemaphoreType.DMA,
        ],
    )
    def k(hist_hbm, s_hbm, r_hbm, ent_hbm, rel_hbm,
          sums_hbm, se_hbm, re_hbm,
          idx_v, rows0, rows1, sums0, sums1, sidx_v, ridx_v, srows_v, rrows_v,
          sem0, sem1, semw0, semw1, sems, semr):
        wid = lax.axis_index("s") * 2 + lax.axis_index("c")
        base_s = wid * SPW
        base_g = wid * GPW

        pltpu.sync_copy(hist_hbm.at[pl.ds(base_g * K, GPW * K)], idx_v)

        pltpu.sync_copy(s_hbm.at[pl.ds(base_s, SPW)], sidx_v)
        pltpu.sync_copy(r_hbm.at[pl.ds(base_s, SPW)], ridx_v)
        se_cp = pltpu.make_async_copy(ent_hbm.at[sidx_v], srows_v, sems)
        re_cp = pltpu.make_async_copy(rel_hbm.at[ridx_v], rrows_v, semr)
        se_cp.start()
        re_cp.start()

        def gather_cp(bi, rows_ref, sem):
            return pltpu.make_async_copy(
                ent_hbm.at[idx_v.at[pl.ds(bi * RB, RB)]], rows_ref, sem)

        def sums_copy(bi, sums_ref, semw):
            return pltpu.make_async_copy(
                sums_ref, sums_hbm.at[pl.ds(base_g + bi * GB, GB)], semw)

        def finish_batch(j, bi, rows_ref, sem, sums_ref, semw):
            gather_cp(bi, rows_ref, sem).wait()

            @pl.when(j > 0)
            def _():
                sums_copy(bi - 2, sums_ref, semw).wait()

            def group_body(g, carry):
                for c in range(C):
                    vs = [rows_ref[g * K + rr, pl.ds(c * LANES, LANES)]
                          for rr in range(K)]
                    while len(vs) > 1:
                        vs = [vs[i] + vs[i + 1] for i in range(0, len(vs), 2)]
                    sums_ref[g, pl.ds(c * LANES, LANES)] = vs[0]
                return carry

            lax.fori_loop(0, GB, group_body, 0)
            sums_copy(bi, sums_ref, semw).start()

        gather_cp(0, rows0, sem0).start()

        def outer(j, carry):
            gather_cp(2 * j + 1, rows1, sem1).start()
            finish_batch(j, 2 * j, rows0, sem0, sums0, semw0)

            @pl.when(j < NBATCH // 2 - 1)
            def _():
                gather_cp(2 * j + 2, rows0, sem0).start()

            finish_batch(j, 2 * j + 1, rows1, sem1, sums1, semw1)
            return carry

        lax.fori_loop(0, NBATCH // 2, outer, 0)

        sums_copy(NBATCH - 2, sums0, semw0).wait()
        sums_copy(NBATCH - 1, sums1, semw1).wait()
        se_cp.wait()
        pltpu.sync_copy(srows_v, se_hbm.at[pl.ds(base_s, SPW)])
        re_cp.wait()
        pltpu.sync_copy(rrows_v, re_hbm.at[pl.ds(base_s, SPW)])

    return k(hist, s, r, ent, rel)


def _tc_finish(sums2, s_e, r_e, W, b2):
    BB = 64

    def body(sums_ref, se_ref, re_ref, w_ref, b_ref, out_ref):
        x = sums_ref[...] * (1.0 / K)
        y = jnp.dot(x, w_ref[...], preferred_element_type=jnp.float32)
        y = jnp.maximum(y + b_ref[...], 0.0)
        out_ref[:, :, 0:H] = y.reshape(BB, T, H)
        out_ref[:, :, H:2 * H] = jnp.broadcast_to(
            se_ref[...][:, None, :], (BB, T, H))
        out_ref[:, :, 2 * H:3 * H] = jnp.broadcast_to(
            re_ref[...][:, None, :], (BB, T, H))

    return pl.pallas_call(
        body,
        grid=(B // BB,),
        in_specs=[
            pl.BlockSpec((BB * T, H), lambda i: (i, 0)),
            pl.BlockSpec((BB, H), lambda i: (i, 0)),
            pl.BlockSpec((BB, H), lambda i: (i, 0)),
            pl.BlockSpec((H, H), lambda i: (0, 0)),
            pl.BlockSpec((1, H), lambda i: (0, 0)),
        ],
        out_specs=pl.BlockSpec((BB, T, 3 * H), lambda i: (i, 0, 0)),
        out_shape=jax.ShapeDtypeStruct((B, T, 3 * H), jnp.float32),
    )(sums2, s_e, r_e, W, b2)


def kernel(s_hist, s, r, ent_embeds, rel_embeds, W, b):
    hist = s_hist.reshape(-1).astype(jnp.int32)
    sums, s_e, r_e = _sc_gather_sum(
        hist, s.astype(jnp.int32), r.astype(jnp.int32),
        ent_embeds, rel_embeds)
    return _tc_finish(sums, s_e, r_e, W, b.reshape(1, H))
